# baseline (speedup 1.0000x reference)
import functools

import jax
import jax.numpy as jnp
from jax import lax
from jax.experimental import pallas as pl
from jax.experimental.pallas import tpu as pltpu
from jax.experimental.pallas import tpu_sc as plsc

_DIM = 64
_PAD = 128
_SCALE = float(_DIM) ** 0.5
_L = 16
_NC, _NS = 2, 16
_NW = _NC * _NS
_BLK = 128


@jax.jit
def _embed_gather(ids_t, table_pad):
    n_slab, n_tok = ids_t.shape
    mesh = plsc.VectorSubcoreMesh(core_axis_name="c", subcore_axis_name="s")

    @functools.partial(
        pl.kernel,
        out_type=jax.ShapeDtypeStruct((n_slab, _DIM, n_tok), jnp.float32),
        mesh=mesh,
        scratch_types=[
            pltpu.VMEM((_BLK,), jnp.int32),
            pltpu.VMEM((_BLK, _PAD), jnp.float32),
            pltpu.VMEM((_DIM, _BLK), jnp.float32),
            pltpu.SemaphoreType.DMA,
        ],
        compiler_params=pltpu.CompilerParams(
            use_tc_tiling_on_sc=True, needs_layout_passes=False
        ),
    )
    def k(ids_hbm, tab_hbm, out_hbm, idx_v, rows_v, blk_v, sem):
        wid = lax.axis_index("s") * _NC + lax.axis_index("c")
        col0 = wid * _BLK

        def do_slab(j, carry):
            pltpu.sync_copy(ids_hbm.at[j, pl.ds(col0, _BLK)], idx_v)
            pltpu.async_copy(tab_hbm.at[idx_v], rows_v, sem).wait()

            def do_dim(d, carry2):
                dcol = jnp.full((_L,), d, jnp.int32)
                for t in range(_BLK // _L):
                    trow = jnp.arange(t * _L, (t + 1) * _L, dtype=jnp.int32)
                    v = plsc.load_gather(rows_v, [trow, dcol])
                    blk_v[d, pl.ds(t * _L, _L)] = v * _SCALE
                return carry2

            lax.fori_loop(0, _DIM, do_dim, 0)
            pltpu.sync_copy(blk_v, out_hbm.at[j, :, pl.ds(col0, _BLK)])
            return carry

        lax.fori_loop(0, n_slab, do_slab, 0)

    return k(ids_t, table_pad)


def kernel(input_ids, table):
    table_pad = jnp.pad(table, ((0, 0), (0, _PAD - _DIM)))
    out3 = _embed_gather(input_ids.T, table_pad)
    return out3.transpose(2, 0, 1)
